# Initial kernel scaffold; baseline (speedup 1.0000x reference)
import jax
import jax.numpy as jnp
from jax.experimental import pallas as pl


def kernel(x):
    raise NotImplementedError("write your pallas kernel here")


# speedup vs baseline: 7.3728x; 7.3728x over previous
import jax
import jax.numpy as jnp
from jax.experimental import pallas as pl
from jax.experimental.pallas import tpu as pltpu

_K = 64
_N = 32768
_ROWS_PER_BLOCK = 16


def _topk_mask_body(x_ref, o_ref):
    x = x_ref[...]
    i = jax.lax.bitcast_convert_type(x, jnp.int32)
    key = jnp.where(i >= 0, i, jnp.bitwise_xor(jnp.bitwise_not(i), jnp.int32(-(2**31))))
    nrows = x.shape[0]
    lo = jnp.full((nrows, 1), jnp.iinfo(jnp.int32).min, jnp.int32)
    hi = jnp.full((nrows, 1), jnp.iinfo(jnp.int32).max, jnp.int32)

    def body(_, carry):
        lo, hi = carry
        mid = (lo >> 1) + (hi >> 1) + (lo & hi & 1)
        cnt = jnp.sum((key >= mid).astype(jnp.float32), axis=1, keepdims=True)
        ge = cnt >= _K
        return jnp.where(ge, mid, lo), jnp.where(ge, hi, mid)

    lo, hi = jax.lax.fori_loop(0, 32, body, (lo, hi))

    is_tie = key == lo
    n_ge = jnp.sum(
        jnp.where(key >= lo, 1.0, 0.0), axis=1, keepdims=True
    ).astype(jnp.int32)
    col = jax.lax.broadcasted_iota(jnp.int32, x.shape, 1)
    tcol = jnp.where(is_tie, col, -1)
    cut = jnp.full((nrows, 1), jnp.iinfo(jnp.int32).max, jnp.int32)
    extra = n_ge - _K
    for _ in range(4):
        hi_col = jnp.max(jnp.where(tcol < cut, tcol, -1), axis=1, keepdims=True)
        cut = jnp.where(extra > 0, hi_col, cut)
        extra = jnp.maximum(extra - 1, 0)

    keep = (key > lo) | (is_tie & (col < cut))
    o_ref[...] = jnp.where(keep, jnp.maximum(x, 0.0), 0.0)


def kernel(x):
    m, n = x.shape
    grid = (m // _ROWS_PER_BLOCK,)
    return pl.pallas_call(
        _topk_mask_body,
        grid=grid,
        in_specs=[pl.BlockSpec((_ROWS_PER_BLOCK, n), lambda r: (r, 0))],
        out_specs=pl.BlockSpec((_ROWS_PER_BLOCK, n), lambda r: (r, 0)),
        out_shape=jax.ShapeDtypeStruct((m, n), x.dtype),
        compiler_params=pltpu.CompilerParams(
            dimension_semantics=("arbitrary",),
        ),
    )(x)
